# baseline (speedup 1.0000x reference)
import functools

import jax
import jax.numpy as jnp
from jax.experimental import pallas as pl
from jax.experimental.pallas import tpu as pltpu

_EPS = 1e-05
_G = 256
_BR = 1000
_D = 480
_NMUL = 224
_NSC = 128


def _mul_of_col(j):
    return jnp.where(
        j < 128, j,
        jnp.where(j < 320, 128 + (j - 128) // 3, 192 + (j - 320) // 5))


def _fused_kernel(nbh, x_ref, b_ref, ms_ref, w_ref, bias_ref, out_ref,
                  xc_ref, stats_ref, ab_ref):
    p = pl.program_id(0)
    i = pl.program_id(1)

    bids = b_ref[0]
    gi = jax.lax.broadcasted_iota(jnp.int32, (_G, _BR), 0)
    onehot = (gi == bids).astype(jnp.float32)

    @pl.when(p == 0)
    def _stats_phase():
        @pl.when(i == 0)
        def _():
            stats_ref[:, :] = jnp.zeros_like(stats_ref)

        x = x_ref[:, :]

        @pl.when(i >= nbh)
        def _():
            xc_ref[pl.ds((i - nbh) * _BR, _BR), :] = x

        ones = jnp.ones((_BR, 1), jnp.float32)
        zpad = jnp.zeros((_BR, 31), jnp.float32)
        vals = jnp.concatenate([ones, x[:, :_NSC], x * x, zpad], axis=1)
        stats_ref[:, :] += jnp.dot(onehot, vals,
                                   preferred_element_type=jnp.float32)

    @pl.when(p == 1)
    def _apply_phase():
        @pl.when(i == 0)
        def _finalize():
            stats = stats_ref[:, :]
            cnt = jnp.maximum(stats[:, 0:1], 1.0)
            fm = stats[:, 1:1 + _NSC] / cnt
            e2 = stats[:, 1 + _NSC:1 + _NSC + _D] / cnt

            jm = jax.lax.broadcasted_iota(jnp.int32, (_D, _NMUL), 0)
            mm = jax.lax.broadcasted_iota(jnp.int32, (_D, _NMUL), 1)
            dinv = jnp.where(mm < 128, 1.0,
                             jnp.where(mm < 192, 1.0 / 3.0, 1.0 / 5.0))
            red = jnp.where(_mul_of_col(jm) == mm, dinv, 0.0)
            norm = jnp.dot(e2, red, preferred_element_type=jnp.float32)

            ms = ms_ref[:, :]
            norm_sc = norm[:, :_NSC] - fm * fm * ms * (2.0 - ms)
            norm = jnp.concatenate([norm_sc, norm[:, _NSC:]], axis=1)
            rstd = jax.lax.rsqrt(norm + _EPS) * w_ref[:, :]

            em = jax.lax.broadcasted_iota(jnp.int32, (_NMUL, _D), 0)
            ej = jax.lax.broadcasted_iota(jnp.int32, (_NMUL, _D), 1)
            exp = (_mul_of_col(ej) == em).astype(jnp.float32)
            a_full = jnp.dot(rstd, exp, preferred_element_type=jnp.float32)
            b_sc = bias_ref[:, :] - fm * ms * rstd[:, :_NSC]

            ab_ref[:, 0:_D] = a_full
            ab_ref[:, _D:512] = jnp.zeros((_G, 512 - _D), jnp.float32)
            ab_ref[:, 512:640] = b_sc

        abn = jax.lax.dot_general(
            onehot, ab_ref[:, :],
            dimension_numbers=(((0,), (0,)), ((), ())),
            preferred_element_type=jnp.float32)

        def _emit(x):
            out = x * abn[:, 0:_D]
            out_ref[:, :] = out
            out_ref[:, 0:_NSC] = out[:, 0:_NSC] + abn[:, 512:640]

        @pl.when(i < nbh)
        def _():
            _emit(x_ref[:, :])

        @pl.when(i >= nbh)
        def _():
            _emit(xc_ref[pl.ds((i - nbh) * _BR, _BR), :])


def kernel(node_input, batch, mean_shift, affine_weight, affine_bias):
    n, d = node_input.shape
    nb = pl.cdiv(n, _BR)
    npad = nb * _BR
    if npad != n:
        x = jnp.pad(node_input, ((0, npad - n), (0, 0)))
        b = jnp.pad(batch.astype(jnp.int32), (0, npad - n), constant_values=_G)
    else:
        x = node_input
        b = batch.astype(jnp.int32)
    b3 = b.reshape(nb, 1, _BR)
    ms2 = mean_shift.reshape(1, _NSC)
    w2 = affine_weight.reshape(1, _NMUL)
    bias2 = affine_bias.reshape(1, _NSC)

    nbh = nb // 2
    out = pl.pallas_call(
        functools.partial(_fused_kernel, nbh),
        grid=(2, nb),
        in_specs=[
            pl.BlockSpec((_BR, _D),
                         lambda p, i: (jnp.where((p == 0) | (i < nbh), i, 0),
                                       0)),
            pl.BlockSpec((1, 1, _BR), lambda p, i: (i, 0, 0)),
            pl.BlockSpec((1, _NSC), lambda p, i: (0, 0)),
            pl.BlockSpec((1, _NMUL), lambda p, i: (0, 0)),
            pl.BlockSpec((1, _NSC), lambda p, i: (0, 0)),
        ],
        out_specs=pl.BlockSpec((_BR, _D),
                               lambda p, i: (jnp.where(p == 0, 0, i), 0)),
        out_shape=jax.ShapeDtypeStruct((npad, _D), jnp.float32),
        scratch_shapes=[
            pltpu.VMEM(((nb - nbh) * _BR, _D), jnp.float32),
            pltpu.VMEM((_G, 640), jnp.float32),
            pltpu.VMEM((_G, 640), jnp.float32),
        ],
        compiler_params=pltpu.CompilerParams(
            vmem_limit_bytes=64 * 1024 * 1024),
    )(x, b3, ms2, w2, bias2)

    return out[:n] if npad != n else out


# speedup vs baseline: 1.1158x; 1.1158x over previous
import functools

import jax
import jax.numpy as jnp
from jax.experimental import pallas as pl
from jax.experimental.pallas import tpu as pltpu

_EPS = 1e-05
_G = 256
_BR = 2000
_CACHE_DTYPE = jnp.bfloat16
_D = 480
_NMUL = 224
_NSC = 128


def _mul_of_col(j):
    return jnp.where(
        j < 128, j,
        jnp.where(j < 320, 128 + (j - 128) // 3, 192 + (j - 320) // 5))


def _fused_kernel(nbh, x_ref, b_ref, ms_ref, w_ref, bias_ref, out_ref,
                  xc_ref, stats_ref, ab_ref):
    p = pl.program_id(0)
    i = pl.program_id(1)

    bids = b_ref[0]
    gi = jax.lax.broadcasted_iota(jnp.int32, (_G, _BR), 0)
    onehot = (gi == bids).astype(jnp.float32)

    @pl.when(p == 0)
    def _stats_phase():
        @pl.when(i == 0)
        def _():
            stats_ref[:, :] = jnp.zeros_like(stats_ref)

        x = x_ref[:, :]

        @pl.when(i >= nbh)
        def _():
            xc_ref[pl.ds((i - nbh) * _BR, _BR), :] = x.astype(_CACHE_DTYPE)

        ones = jnp.ones((_BR, 1), jnp.float32)
        zpad = jnp.zeros((_BR, 31), jnp.float32)
        vals = jnp.concatenate([ones, x[:, :_NSC], x * x, zpad], axis=1)
        stats_ref[:, :] += jnp.dot(onehot, vals,
                                   preferred_element_type=jnp.float32)

    @pl.when(p == 1)
    def _apply_phase():
        @pl.when(i == 0)
        def _finalize():
            stats = stats_ref[:, :]
            cnt = jnp.maximum(stats[:, 0:1], 1.0)
            fm = stats[:, 1:1 + _NSC] / cnt
            e2 = stats[:, 1 + _NSC:1 + _NSC + _D] / cnt

            jm = jax.lax.broadcasted_iota(jnp.int32, (_D, _NMUL), 0)
            mm = jax.lax.broadcasted_iota(jnp.int32, (_D, _NMUL), 1)
            dinv = jnp.where(mm < 128, 1.0,
                             jnp.where(mm < 192, 1.0 / 3.0, 1.0 / 5.0))
            red = jnp.where(_mul_of_col(jm) == mm, dinv, 0.0)
            norm = jnp.dot(e2, red, preferred_element_type=jnp.float32)

            ms = ms_ref[:, :]
            norm_sc = norm[:, :_NSC] - fm * fm * ms * (2.0 - ms)
            norm = jnp.concatenate([norm_sc, norm[:, _NSC:]], axis=1)
            rstd = jax.lax.rsqrt(norm + _EPS) * w_ref[:, :]

            em = jax.lax.broadcasted_iota(jnp.int32, (_NMUL, _D), 0)
            ej = jax.lax.broadcasted_iota(jnp.int32, (_NMUL, _D), 1)
            exp = (_mul_of_col(ej) == em).astype(jnp.float32)
            a_full = jnp.dot(rstd, exp, preferred_element_type=jnp.float32)
            b_sc = bias_ref[:, :] - fm * ms * rstd[:, :_NSC]

            ab_ref[:, 0:_D] = a_full
            ab_ref[:, _D:512] = jnp.zeros((_G, 512 - _D), jnp.float32)
            ab_ref[:, 512:640] = b_sc

        abn = jax.lax.dot_general(
            onehot, ab_ref[:, :],
            dimension_numbers=(((0,), (0,)), ((), ())),
            preferred_element_type=jnp.float32)

        def _emit(x):
            out = x * abn[:, 0:_D]
            out_ref[:, :] = out
            out_ref[:, 0:_NSC] = out[:, 0:_NSC] + abn[:, 512:640]

        @pl.when(i < nbh)
        def _():
            _emit(x_ref[:, :])

        @pl.when(i >= nbh)
        def _():
            _emit(xc_ref[pl.ds((i - nbh) * _BR, _BR), :].astype(jnp.float32))


def kernel(node_input, batch, mean_shift, affine_weight, affine_bias):
    n, d = node_input.shape
    nb = pl.cdiv(n, _BR)
    npad = nb * _BR
    if npad != n:
        x = jnp.pad(node_input, ((0, npad - n), (0, 0)))
        b = jnp.pad(batch.astype(jnp.int32), (0, npad - n), constant_values=_G)
    else:
        x = node_input
        b = batch.astype(jnp.int32)
    b3 = b.reshape(nb, 1, _BR)
    ms2 = mean_shift.reshape(1, _NSC)
    w2 = affine_weight.reshape(1, _NMUL)
    bias2 = affine_bias.reshape(1, _NSC)

    nbh = min(6, nb)
    out = pl.pallas_call(
        functools.partial(_fused_kernel, nbh),
        grid=(2, nb),
        in_specs=[
            pl.BlockSpec((_BR, _D),
                         lambda p, i: (jnp.where((p == 0) | (i < nbh), i, 0),
                                       0)),
            pl.BlockSpec((1, 1, _BR), lambda p, i: (i, 0, 0)),
            pl.BlockSpec((1, _NSC), lambda p, i: (0, 0)),
            pl.BlockSpec((1, _NMUL), lambda p, i: (0, 0)),
            pl.BlockSpec((1, _NSC), lambda p, i: (0, 0)),
        ],
        out_specs=pl.BlockSpec((_BR, _D),
                               lambda p, i: (jnp.where(p == 0, 0, i), 0)),
        out_shape=jax.ShapeDtypeStruct((npad, _D), jnp.float32),
        scratch_shapes=[
            pltpu.VMEM(((nb - nbh) * _BR, _D), _CACHE_DTYPE),
            pltpu.VMEM((_G, 640), jnp.float32),
            pltpu.VMEM((_G, 640), jnp.float32),
        ],
        compiler_params=pltpu.CompilerParams(
            vmem_limit_bytes=64 * 1024 * 1024),
    )(x, b3, ms2, w2, bias2)

    return out[:n] if npad != n else out


# speedup vs baseline: 1.1164x; 1.0005x over previous
import functools

import jax
import jax.numpy as jnp
from jax.experimental import pallas as pl
from jax.experimental.pallas import tpu as pltpu

_EPS = 1e-05
_G = 256
_BR = 2000
_CACHE_DTYPE = jnp.bfloat16
_D = 480
_NMUL = 224
_NSC = 128


def _mul_of_col(j):
    return jnp.where(
        j < 128, j,
        jnp.where(j < 320, 128 + (j - 128) // 3, 192 + (j - 320) // 5))


def _fused_kernel(nbh, nb, x_ref, b_ref, ms_ref, w_ref, bias_ref, out_ref,
                  xc_ref, stats_ref, ab_ref):
    p = pl.program_id(0)
    i = pl.program_id(1)

    bids = b_ref[0]
    gi = jax.lax.broadcasted_iota(jnp.int32, (_G, _BR), 0)
    onehot = (gi == bids).astype(jnp.float32)

    @pl.when(p == 0)
    def _stats_phase():
        @pl.when(i == 0)
        def _():
            stats_ref[:, :] = jnp.zeros_like(stats_ref)

        x = x_ref[:, :]

        @pl.when(i >= nbh)
        def _():
            xc_ref[pl.ds((i - nbh) * _BR, _BR), :] = x.astype(_CACHE_DTYPE)

        ones = jnp.ones((_BR, 1), jnp.float32)
        zpad = jnp.zeros((_BR, 31), jnp.float32)
        vals = jnp.concatenate([ones, x[:, :_NSC], x * x, zpad], axis=1)
        stats_ref[:, :] += jnp.dot(onehot, vals,
                                   preferred_element_type=jnp.float32)

        @pl.when(i == nb - 1)
        def _finalize():
            stats = stats_ref[:, :]
            cnt = jnp.maximum(stats[:, 0:1], 1.0)
            fm = stats[:, 1:1 + _NSC] / cnt
            e2 = stats[:, 1 + _NSC:1 + _NSC + _D] / cnt

            jm = jax.lax.broadcasted_iota(jnp.int32, (_D, _NMUL), 0)
            mm = jax.lax.broadcasted_iota(jnp.int32, (_D, _NMUL), 1)
            dinv = jnp.where(mm < 128, 1.0,
                             jnp.where(mm < 192, 1.0 / 3.0, 1.0 / 5.0))
            red = jnp.where(_mul_of_col(jm) == mm, dinv, 0.0)
            norm = jnp.dot(e2, red, preferred_element_type=jnp.float32)

            ms = ms_ref[:, :]
            norm_sc = norm[:, :_NSC] - fm * fm * ms * (2.0 - ms)
            norm = jnp.concatenate([norm_sc, norm[:, _NSC:]], axis=1)
            rstd = jax.lax.rsqrt(norm + _EPS) * w_ref[:, :]

            em = jax.lax.broadcasted_iota(jnp.int32, (_NMUL, _D), 0)
            ej = jax.lax.broadcasted_iota(jnp.int32, (_NMUL, _D), 1)
            exp = (_mul_of_col(ej) == em).astype(jnp.float32)
            a_full = jnp.dot(rstd, exp, preferred_element_type=jnp.float32)
            b_sc = bias_ref[:, :] - fm * ms * rstd[:, :_NSC]

            ab_ref[:, 0:_D] = a_full
            ab_ref[:, _D:512] = jnp.zeros((_G, 512 - _D), jnp.float32)
            ab_ref[:, 512:640] = b_sc

    @pl.when(p == 1)
    def _apply_phase():
        abn = jax.lax.dot_general(
            onehot, ab_ref[:, :],
            dimension_numbers=(((0,), (0,)), ((), ())),
            preferred_element_type=jnp.float32)

        def _emit(x):
            out = x * abn[:, 0:_D]
            out_ref[:, :] = out
            out_ref[:, 0:_NSC] = out[:, 0:_NSC] + abn[:, 512:640]

        @pl.when(i < nbh)
        def _():
            _emit(x_ref[:, :])

        @pl.when(i >= nbh)
        def _():
            _emit(xc_ref[pl.ds((i - nbh) * _BR, _BR), :].astype(jnp.float32))


def kernel(node_input, batch, mean_shift, affine_weight, affine_bias):
    n, d = node_input.shape
    nb = pl.cdiv(n, _BR)
    npad = nb * _BR
    if npad != n:
        x = jnp.pad(node_input, ((0, npad - n), (0, 0)))
        b = jnp.pad(batch.astype(jnp.int32), (0, npad - n), constant_values=_G)
    else:
        x = node_input
        b = batch.astype(jnp.int32)
    b3 = b.reshape(nb, 1, _BR)
    ms2 = mean_shift.reshape(1, _NSC)
    w2 = affine_weight.reshape(1, _NMUL)
    bias2 = affine_bias.reshape(1, _NSC)

    nbh = min(6, nb)
    out = pl.pallas_call(
        functools.partial(_fused_kernel, nbh, nb),
        grid=(2, nb),
        in_specs=[
            pl.BlockSpec((_BR, _D),
                         lambda p, i: (jnp.where((p == 0) | (i < nbh), i, 0),
                                       0)),
            pl.BlockSpec((1, 1, _BR), lambda p, i: (i, 0, 0)),
            pl.BlockSpec((1, _NSC), lambda p, i: (0, 0)),
            pl.BlockSpec((1, _NMUL), lambda p, i: (0, 0)),
            pl.BlockSpec((1, _NSC), lambda p, i: (0, 0)),
        ],
        out_specs=pl.BlockSpec((_BR, _D),
                               lambda p, i: (jnp.where(p == 0, 0, i), 0)),
        out_shape=jax.ShapeDtypeStruct((npad, _D), jnp.float32),
        scratch_shapes=[
            pltpu.VMEM(((nb - nbh) * _BR, _D), _CACHE_DTYPE),
            pltpu.VMEM((_G, 640), jnp.float32),
            pltpu.VMEM((_G, 640), jnp.float32),
        ],
        compiler_params=pltpu.CompilerParams(
            vmem_limit_bytes=64 * 1024 * 1024),
    )(x, b3, ms2, w2, bias2)

    return out[:n] if npad != n else out
